# baseline (speedup 1.0000x reference)
import jax
import jax.numpy as jnp
from jax import lax
from jax.experimental import pallas as pl
from jax.experimental.pallas import tpu as pltpu
from jax.experimental.pallas import tpu_sc as plsc

_N = 10000
_NP = 10240
_D = 128
_W = 144
_DO = 16
_E = 320000
_CH = 64
_NC = 2
_NS = 16
_NW = _NC * _NS
_NCHUNK = 2 * (-(-_E // (2 * _NW * _CH)))
_EP = _NCHUNK * _NW * _CH
_EPA = _EP + 2 * _CH
_RPT = _NP // _NS


def _sc_conv_body(xe, src, dst, beta, z144,
                  acc_out,
                  acc_sh, srcv0, srcv1, dstv0, dstv1, dsc0, dsc1,
                  S0, S1, D0, D1, fv, betav,
                  semS0, semS1, semD0, semD1, semSc0, semSc1,
                  semIs0, semIs1, semId0, semId1):
    c = lax.axis_index("c")
    s = lax.axis_index("s")
    w = c * _NS + s
    r0 = s * _RPT
    srcv = (srcv0, srcv1)
    dstv = (dstv0, dstv1)
    dsc = (dsc0, dsc1)
    S = (S0, S1)
    Dv = (D0, D1)
    semS = (semS0, semS1)
    semD = (semD0, semD1)
    semSc = (semSc0, semSc1)
    semIs = (semIs0, semIs1)
    semId = (semId0, semId1)
    pltpu.sync_copy(z144.at[pl.ds(r0, _RPT)], acc_sh.at[pl.ds(r0, _RPT)])
    pltpu.sync_copy(beta, betav)
    bvec = betav[...]
    plsc.subcore_barrier()

    ebase = w * (_NCHUNK * _CH)

    def body(g, b, first):
        nb = 1 - b
        pltpu.make_async_copy(xe.at[srcv[b]], S[b], semS[b]).wait()
        pltpu.make_async_copy(xe.at[dstv[b]], Dv[b], semD[b]).wait()
        if not first:
            pltpu.make_async_copy(S[nb], acc_sh.at[dsc[nb]], semSc[nb]).wait()
        pltpu.make_async_copy(src.at[pl.ds(0, _CH)], srcv[nb], semIs[nb]).wait()
        pltpu.make_async_copy(dst.at[pl.ds(0, _CH)], dstv[nb], semId[nb]).wait()
        pltpu.async_copy(xe.at[srcv[nb]], S[nb], semS[nb])
        pltpu.async_copy(xe.at[dstv[nb]], Dv[nb], semD[nb])

        Sb, Db = S[b], Dv[b]

        def edge_dot(e, u):
            p = [Sb[e, pl.ds(16 * j, 16)] * Db[e, pl.ds(16 * j, 16)]
                 for j in range(_D // 16)]
            while len(p) > 1:
                p = [p[i] + p[i + 1] for i in range(0, len(p), 2)]
            alpha = jnp.sum(p[0])
            ea16 = jnp.exp(bvec * alpha)
            fv[e, :] = ea16 * Sb[e, pl.ds(_D, 16)][0]
            Sb[e, pl.ds(_D, 16)] = ea16
            return u

        def edge_scale(e, u):
            f16 = fv[e, :]
            for j in range(_D // 16):
                Sb[e, pl.ds(16 * j, 16)] = Sb[e, pl.ds(16 * j, 16)] * f16
            return u

        lax.fori_loop(0, _CH, edge_dot, 0, unroll=8)
        lax.fori_loop(0, _CH, edge_scale, 0, unroll=4)
        for i in range(_CH // 16):
            dsc[b][pl.ds(16 * i, 16)] = dstv[b][pl.ds(16 * i, 16)]
        pltpu.async_copy(S[b], acc_sh.at[dsc[b]], semSc[b], add=True)
        base2 = ebase + (g + 2) * _CH
        pltpu.async_copy(src.at[pl.ds(base2, _CH)], srcv[b], semIs[b])
        pltpu.async_copy(dst.at[pl.ds(base2, _CH)], dstv[b], semId[b])

    ci0 = pltpu.async_copy(src.at[pl.ds(ebase, _CH)], srcv0, semIs0)
    ci1 = pltpu.async_copy(dst.at[pl.ds(ebase, _CH)], dstv0, semId0)
    pltpu.async_copy(src.at[pl.ds(ebase + _CH, _CH)], srcv1, semIs1)
    pltpu.async_copy(dst.at[pl.ds(ebase + _CH, _CH)], dstv1, semId1)
    ci0.wait()
    ci1.wait()
    pltpu.async_copy(xe.at[srcv0], S0, semS0)
    pltpu.async_copy(xe.at[dstv0], D0, semD0)

    body(0, 0, True)

    def two_chunks(gg, carry):
        for i in range(2):
            body(2 * gg + 1 + i, (1 + i) % 2, False)
        return carry

    lax.fori_loop(0, (_NCHUNK - 2) // 2, two_chunks, 0)
    body(_NCHUNK - 1, 1, False)
    pltpu.make_async_copy(xe.at[srcv0], S0, semS0).wait()
    pltpu.make_async_copy(xe.at[dstv0], D0, semD0).wait()
    pltpu.make_async_copy(S1, acc_sh.at[dsc1], semSc1).wait()
    pltpu.make_async_copy(src.at[pl.ds(0, _CH)], srcv1, semIs1).wait()
    pltpu.make_async_copy(dst.at[pl.ds(0, _CH)], dstv1, semId1).wait()
    plsc.subcore_barrier()
    pltpu.sync_copy(acc_sh.at[pl.ds(r0, _RPT)], acc_out.at[c, pl.ds(r0, _RPT)])


_sc_conv = pl.kernel(
    _sc_conv_body,
    out_type=jax.ShapeDtypeStruct((_NC, _NP, _W), jnp.float32),
    mesh=plsc.VectorSubcoreMesh(core_axis_name="c", subcore_axis_name="s"),
    compiler_params=pltpu.CompilerParams(needs_layout_passes=False,
                                         use_tc_tiling_on_sc=False),
    scratch_types=[
        pltpu.VMEM_SHARED((_NP, _W), jnp.float32),
        pltpu.VMEM((_CH,), jnp.int32),
        pltpu.VMEM((_CH,), jnp.int32),
        pltpu.VMEM((_CH,), jnp.int32),
        pltpu.VMEM((_CH,), jnp.int32),
        pltpu.VMEM((_CH,), jnp.int32),
        pltpu.VMEM((_CH,), jnp.int32),
        pltpu.VMEM((_CH, _W), jnp.float32),
        pltpu.VMEM((_CH, _W), jnp.float32),
        pltpu.VMEM((_CH, _W), jnp.float32),
        pltpu.VMEM((_CH, _W), jnp.float32),
        pltpu.VMEM((_CH, 16), jnp.float32),
        pltpu.VMEM((16,), jnp.float32),
    ] + [pltpu.SemaphoreType.DMA] * 10,
)


def _table(h):
    n = jnp.sqrt(jnp.sum(h * h, axis=1, keepdims=True))
    hn = h / jnp.maximum(n, 1e-12)
    return jnp.concatenate(
        [hn, n, jnp.zeros((_NP, _W - _D - 1), jnp.float32)], axis=1)


def _pre_body(x_ref, w_ref, b_ref, xe_ref):
    h = jnp.dot(x_ref[...], w_ref[...], preferred_element_type=jnp.float32)
    xe_ref[...] = _table(jnp.maximum(h + b_ref[...], 0.0))


_pre_call = pl.pallas_call(
    _pre_body,
    out_shape=jax.ShapeDtypeStruct((_NP, _W), jnp.float32),
)


def _combine(acc_ref, xe_ref, beta):
    n = xe_ref[:, _D:_D + 1]
    sdot = (n / jnp.maximum(n, 1e-12)) ** 2
    es = jnp.exp(beta * sdot)
    num = (acc_ref[0, :, :_D] + acc_ref[1, :, :_D]
           + es * (n * xe_ref[:, :_D]))
    den = (acc_ref[0, :, _D:_D + 1] + acc_ref[1, :, _D:_D + 1]
           + es + 1e-16)
    return num / den


def _mid_body(acc_ref, xe_ref, xe2_ref):
    xe2_ref[...] = _table(_combine(acc_ref, xe_ref, 1.0))


_mid_call = pl.pallas_call(
    _mid_body,
    out_shape=jax.ShapeDtypeStruct((_NP, _W), jnp.float32),
)


def _post_body(acc_ref, xe_ref, beta_ref, w_ref, b_ref, out_ref):
    h1 = _combine(acc_ref, xe_ref, beta_ref[0, 0])
    o = jnp.dot(h1, w_ref[...], preferred_element_type=jnp.float32) + b_ref[...]
    m = jnp.max(o, axis=1, keepdims=True)
    lse = jnp.log(jnp.sum(jnp.exp(o - m), axis=1, keepdims=True)) + m
    out_ref[...] = o - lse


_post_call = pl.pallas_call(
    _post_body,
    out_shape=jax.ShapeDtypeStruct((_NP, _DO), jnp.float32),
)


def kernel(x, edge_index, W1, b1, beta2, W2, b2):
    src = edge_index[0].astype(jnp.int32)
    dst = edge_index[1].astype(jnp.int32)
    dummy = jnp.full((_EPA - _E,), _N, jnp.int32)
    src = jnp.concatenate([src, dummy])
    dst = jnp.concatenate([dst, dummy])
    xp = jnp.zeros((_NP, _D), jnp.float32).at[:_N].set(x.astype(jnp.float32))

    z144 = jnp.zeros((_NP, _W), jnp.float32)
    beta2f = beta2.astype(jnp.float32)

    xe1 = _pre_call(xp, W1.T, b1.reshape(1, _D))
    acc1 = _sc_conv(xe1, src, dst, jnp.ones((16,), jnp.float32), z144)
    xe2 = _mid_call(acc1, xe1)
    acc2 = _sc_conv(xe2, src, dst, jnp.full((16,), beta2f, jnp.float32), z144)
    out = _post_call(acc2, xe2, beta2f.reshape(1, 1), W2.T, b2.reshape(1, _DO))
    return out[:_N]


# speedup vs baseline: 1.0678x; 1.0678x over previous
import jax
import jax.numpy as jnp
from jax import lax
from jax.experimental import pallas as pl
from jax.experimental.pallas import tpu as pltpu
from jax.experimental.pallas import tpu_sc as plsc

_N = 10000
_NP = 10240
_D = 128
_W = 144
_DO = 16
_E = 320000
_CH = 64
_NC = 2
_NS = 16
_NW = _NC * _NS
_NCHUNK = 2 * (-(-_E // (2 * _NW * _CH)))
_EP = _NCHUNK * _NW * _CH
_EPA = _EP + 2 * _CH
_RPT = _NP // _NS


def _sc_conv_body(xe, src, dst, beta, z144,
                  acc_out,
                  acc_sh, srcv0, srcv1, dstv0, dstv1, dsc0, dsc1,
                  S0, S1, D0, D1, fv, betav,
                  semS0, semS1, semD0, semD1, semSc0, semSc1,
                  semIs0, semIs1, semId0, semId1):
    c = lax.axis_index("c")
    s = lax.axis_index("s")
    w = c * _NS + s
    r0 = s * _RPT
    srcv = (srcv0, srcv1)
    dstv = (dstv0, dstv1)
    dsc = (dsc0, dsc1)
    S = (S0, S1)
    Dv = (D0, D1)
    semS = (semS0, semS1)
    semD = (semD0, semD1)
    semSc = (semSc0, semSc1)
    semIs = (semIs0, semIs1)
    semId = (semId0, semId1)
    pltpu.sync_copy(z144.at[pl.ds(r0, _RPT)], acc_sh.at[pl.ds(r0, _RPT)])
    pltpu.sync_copy(beta, betav)
    bvec = betav[...]
    plsc.subcore_barrier()

    ebase = w * (_NCHUNK * _CH)

    def body(g, b, first):
        nb = 1 - b
        pltpu.make_async_copy(xe.at[srcv[b]], S[b], semS[b]).wait()
        pltpu.make_async_copy(xe.at[dstv[b]], Dv[b], semD[b]).wait()
        if not first:
            pltpu.make_async_copy(S[nb], acc_sh.at[dsc[nb]], semSc[nb]).wait()
        pltpu.make_async_copy(src.at[pl.ds(0, _CH)], srcv[nb], semIs[nb]).wait()
        pltpu.make_async_copy(dst.at[pl.ds(0, _CH)], dstv[nb], semId[nb]).wait()
        pltpu.async_copy(xe.at[srcv[nb]], S[nb], semS[nb])
        pltpu.async_copy(xe.at[dstv[nb]], Dv[nb], semD[nb])

        Sb, Db = S[b], Dv[b]

        def edge(e, u):
            sv = [Sb[e, pl.ds(16 * j, 16)] for j in range(_D // 16)]
            p = [sv[j] * Db[e, pl.ds(16 * j, 16)] for j in range(_D // 16)]
            while len(p) > 1:
                p = [p[i] + p[i + 1] for i in range(0, len(p), 2)]
            alpha = jnp.sum(p[0])
            ea16 = jnp.exp(bvec * alpha)
            f16 = ea16 * Sb[e, pl.ds(_D, 16)][0]
            for j in range(_D // 16):
                Sb[e, pl.ds(16 * j, 16)] = sv[j] * f16
            Sb[e, pl.ds(_D, 16)] = ea16
            return u

        lax.fori_loop(0, _CH, edge, 0, unroll=6)
        for i in range(_CH // 16):
            dsc[b][pl.ds(16 * i, 16)] = dstv[b][pl.ds(16 * i, 16)]
        pltpu.async_copy(S[b], acc_sh.at[dsc[b]], semSc[b], add=True)
        base2 = ebase + (g + 2) * _CH
        pltpu.async_copy(src.at[pl.ds(base2, _CH)], srcv[b], semIs[b])
        pltpu.async_copy(dst.at[pl.ds(base2, _CH)], dstv[b], semId[b])

    ci0 = pltpu.async_copy(src.at[pl.ds(ebase, _CH)], srcv0, semIs0)
    ci1 = pltpu.async_copy(dst.at[pl.ds(ebase, _CH)], dstv0, semId0)
    pltpu.async_copy(src.at[pl.ds(ebase + _CH, _CH)], srcv1, semIs1)
    pltpu.async_copy(dst.at[pl.ds(ebase + _CH, _CH)], dstv1, semId1)
    ci0.wait()
    ci1.wait()
    pltpu.async_copy(xe.at[srcv0], S0, semS0)
    pltpu.async_copy(xe.at[dstv0], D0, semD0)

    body(0, 0, True)

    def two_chunks(gg, carry):
        for i in range(2):
            body(2 * gg + 1 + i, (1 + i) % 2, False)
        return carry

    lax.fori_loop(0, (_NCHUNK - 2) // 2, two_chunks, 0)
    body(_NCHUNK - 1, 1, False)
    pltpu.make_async_copy(xe.at[srcv0], S0, semS0).wait()
    pltpu.make_async_copy(xe.at[dstv0], D0, semD0).wait()
    pltpu.make_async_copy(S1, acc_sh.at[dsc1], semSc1).wait()
    pltpu.make_async_copy(src.at[pl.ds(0, _CH)], srcv1, semIs1).wait()
    pltpu.make_async_copy(dst.at[pl.ds(0, _CH)], dstv1, semId1).wait()
    plsc.subcore_barrier()
    pltpu.sync_copy(acc_sh.at[pl.ds(r0, _RPT)], acc_out.at[c, pl.ds(r0, _RPT)])


_sc_conv = pl.kernel(
    _sc_conv_body,
    out_type=jax.ShapeDtypeStruct((_NC, _NP, _W), jnp.float32),
    mesh=plsc.VectorSubcoreMesh(core_axis_name="c", subcore_axis_name="s"),
    compiler_params=pltpu.CompilerParams(needs_layout_passes=False,
                                         use_tc_tiling_on_sc=False),
    scratch_types=[
        pltpu.VMEM_SHARED((_NP, _W), jnp.float32),
        pltpu.VMEM((_CH,), jnp.int32),
        pltpu.VMEM((_CH,), jnp.int32),
        pltpu.VMEM((_CH,), jnp.int32),
        pltpu.VMEM((_CH,), jnp.int32),
        pltpu.VMEM((_CH,), jnp.int32),
        pltpu.VMEM((_CH,), jnp.int32),
        pltpu.VMEM((_CH, _W), jnp.float32),
        pltpu.VMEM((_CH, _W), jnp.float32),
        pltpu.VMEM((_CH, _W), jnp.float32),
        pltpu.VMEM((_CH, _W), jnp.float32),
        pltpu.VMEM((_CH, 16), jnp.float32),
        pltpu.VMEM((16,), jnp.float32),
    ] + [pltpu.SemaphoreType.DMA] * 10,
)


def _table(h):
    n = jnp.sqrt(jnp.sum(h * h, axis=1, keepdims=True))
    hn = h / jnp.maximum(n, 1e-12)
    return jnp.concatenate(
        [hn, n, jnp.zeros((_NP, _W - _D - 1), jnp.float32)], axis=1)


def _pre_body(x_ref, w_ref, b_ref, xe_ref):
    h = jnp.dot(x_ref[...], w_ref[...], preferred_element_type=jnp.float32)
    xe_ref[...] = _table(jnp.maximum(h + b_ref[...], 0.0))


_pre_call = pl.pallas_call(
    _pre_body,
    out_shape=jax.ShapeDtypeStruct((_NP, _W), jnp.float32),
)


def _combine(acc_ref, xe_ref, beta):
    n = xe_ref[:, _D:_D + 1]
    sdot = (n / jnp.maximum(n, 1e-12)) ** 2
    es = jnp.exp(beta * sdot)
    num = (acc_ref[0, :, :_D] + acc_ref[1, :, :_D]
           + es * (n * xe_ref[:, :_D]))
    den = (acc_ref[0, :, _D:_D + 1] + acc_ref[1, :, _D:_D + 1]
           + es + 1e-16)
    return num / den


def _mid_body(acc_ref, xe_ref, xe2_ref):
    xe2_ref[...] = _table(_combine(acc_ref, xe_ref, 1.0))


_mid_call = pl.pallas_call(
    _mid_body,
    out_shape=jax.ShapeDtypeStruct((_NP, _W), jnp.float32),
)


def _post_body(acc_ref, xe_ref, beta_ref, w_ref, b_ref, out_ref):
    h1 = _combine(acc_ref, xe_ref, beta_ref[0, 0])
    o = jnp.dot(h1, w_ref[...], preferred_element_type=jnp.float32) + b_ref[...]
    m = jnp.max(o, axis=1, keepdims=True)
    lse = jnp.log(jnp.sum(jnp.exp(o - m), axis=1, keepdims=True)) + m
    out_ref[...] = o - lse


_post_call = pl.pallas_call(
    _post_body,
    out_shape=jax.ShapeDtypeStruct((_NP, _DO), jnp.float32),
)


def kernel(x, edge_index, W1, b1, beta2, W2, b2):
    src = edge_index[0].astype(jnp.int32)
    dst = edge_index[1].astype(jnp.int32)
    dummy = jnp.full((_EPA - _E,), _N, jnp.int32)
    src = jnp.concatenate([src, dummy])
    dst = jnp.concatenate([dst, dummy])
    xp = jnp.zeros((_NP, _D), jnp.float32).at[:_N].set(x.astype(jnp.float32))

    z144 = jnp.zeros((_NP, _W), jnp.float32)
    beta2f = beta2.astype(jnp.float32)

    xe1 = _pre_call(xp, W1.T, b1.reshape(1, _D))
    acc1 = _sc_conv(xe1, src, dst, jnp.ones((16,), jnp.float32), z144)
    xe2 = _mid_call(acc1, xe1)
    acc2 = _sc_conv(xe2, src, dst, jnp.full((16,), beta2f, jnp.float32), z144)
    out = _post_call(acc2, xe2, beta2f.reshape(1, 1), W2.T, b2.reshape(1, _DO))
    return out[:_N]


# speedup vs baseline: 1.4440x; 1.3524x over previous
import jax
import jax.numpy as jnp
from jax import lax
from jax.experimental import pallas as pl
from jax.experimental.pallas import tpu as pltpu
from jax.experimental.pallas import tpu_sc as plsc

_N = 10000
_NP = 10240
_D = 128
_W = 144
_DO = 16
_E = 320000
_CH = 64
_NC = 2
_NS = 16
_NW = _NC * _NS
_NCHUNK = 2 * (-(-_E // (2 * _NW * _CH)))
_EP = _NCHUNK * _NW * _CH
_EPA = _EP + 2 * _CH
_RPT = _NP // _NS


def _sc_conv_body(xe, xn, src, dst, beta, z144,
                  acc_out,
                  acc_sh, srcv0, srcv1, dstv0, dstv1, dsc0, dsc1,
                  S0, S1, D0, D1, fv, betav,
                  semS0, semS1, semD0, semD1, semSc0, semSc1,
                  semIs0, semIs1, semId0, semId1):
    c = lax.axis_index("c")
    s = lax.axis_index("s")
    w = c * _NS + s
    r0 = s * _RPT
    srcv = (srcv0, srcv1)
    dstv = (dstv0, dstv1)
    dsc = (dsc0, dsc1)
    S = (S0, S1)
    Dv = (D0, D1)
    semS = (semS0, semS1)
    semD = (semD0, semD1)
    semSc = (semSc0, semSc1)
    semIs = (semIs0, semIs1)
    semId = (semId0, semId1)
    pltpu.sync_copy(z144.at[pl.ds(r0, _RPT)], acc_sh.at[pl.ds(r0, _RPT)])
    pltpu.sync_copy(beta, betav)
    bvec = betav[...]
    plsc.subcore_barrier()

    ebase = w * (_NCHUNK * _CH)

    def body(g, b, first):
        nb = 1 - b
        pltpu.make_async_copy(xe.at[srcv[b]], S[b], semS[b]).wait()
        pltpu.make_async_copy(xn.at[dstv[b]], Dv[b], semD[b]).wait()
        if not first:
            pltpu.make_async_copy(S[nb], acc_sh.at[dsc[nb]], semSc[nb]).wait()
        pltpu.make_async_copy(src.at[pl.ds(0, _CH)], srcv[nb], semIs[nb]).wait()
        pltpu.make_async_copy(dst.at[pl.ds(0, _CH)], dstv[nb], semId[nb]).wait()
        pltpu.async_copy(xe.at[srcv[nb]], S[nb], semS[nb])
        pltpu.async_copy(xn.at[dstv[nb]], Dv[nb], semD[nb])

        Sb, Db = S[b], Dv[b]

        def edge(e, u):
            sv = [Sb[e, pl.ds(16 * j, 16)] for j in range(_D // 16)]
            p = [sv[j] * Db[e, pl.ds(16 * j, 16)] for j in range(_D // 16)]
            while len(p) > 1:
                p = [p[i] + p[i + 1] for i in range(0, len(p), 2)]
            alpha = jnp.sum(p[0])
            ea16 = jnp.exp(bvec * alpha)
            f16 = ea16 * Sb[e, pl.ds(_D, 16)][0]
            for j in range(_D // 16):
                Sb[e, pl.ds(16 * j, 16)] = sv[j] * f16
            Sb[e, pl.ds(_D, 16)] = ea16
            return u

        lax.fori_loop(0, _CH, edge, 0, unroll=6)
        for i in range(_CH // 16):
            dsc[b][pl.ds(16 * i, 16)] = dstv[b][pl.ds(16 * i, 16)]
        pltpu.async_copy(S[b], acc_sh.at[dsc[b]], semSc[b], add=True)
        base2 = ebase + (g + 2) * _CH
        pltpu.async_copy(src.at[pl.ds(base2, _CH)], srcv[b], semIs[b])
        pltpu.async_copy(dst.at[pl.ds(base2, _CH)], dstv[b], semId[b])

    ci0 = pltpu.async_copy(src.at[pl.ds(ebase, _CH)], srcv0, semIs0)
    ci1 = pltpu.async_copy(dst.at[pl.ds(ebase, _CH)], dstv0, semId0)
    pltpu.async_copy(src.at[pl.ds(ebase + _CH, _CH)], srcv1, semIs1)
    pltpu.async_copy(dst.at[pl.ds(ebase + _CH, _CH)], dstv1, semId1)
    ci0.wait()
    ci1.wait()
    pltpu.async_copy(xe.at[srcv0], S0, semS0)
    pltpu.async_copy(xn.at[dstv0], D0, semD0)

    body(0, 0, True)

    def two_chunks(gg, carry):
        for i in range(2):
            body(2 * gg + 1 + i, (1 + i) % 2, False)
        return carry

    lax.fori_loop(0, (_NCHUNK - 2) // 2, two_chunks, 0)
    body(_NCHUNK - 1, 1, False)
    pltpu.make_async_copy(xe.at[srcv0], S0, semS0).wait()
    pltpu.make_async_copy(xn.at[dstv0], D0, semD0).wait()
    pltpu.make_async_copy(S1, acc_sh.at[dsc1], semSc1).wait()
    pltpu.make_async_copy(src.at[pl.ds(0, _CH)], srcv1, semIs1).wait()
    pltpu.make_async_copy(dst.at[pl.ds(0, _CH)], dstv1, semId1).wait()
    plsc.subcore_barrier()
    pltpu.sync_copy(acc_sh.at[pl.ds(r0, _RPT)], acc_out.at[c, pl.ds(r0, _RPT)])


_sc_conv = pl.kernel(
    _sc_conv_body,
    out_type=jax.ShapeDtypeStruct((_NC, _NP, _W), jnp.float32),
    mesh=plsc.VectorSubcoreMesh(core_axis_name="c", subcore_axis_name="s"),
    compiler_params=pltpu.CompilerParams(needs_layout_passes=False,
                                         use_tc_tiling_on_sc=False),
    scratch_types=[
        pltpu.VMEM_SHARED((_NP, _W), jnp.float32),
        pltpu.VMEM((_CH,), jnp.int32),
        pltpu.VMEM((_CH,), jnp.int32),
        pltpu.VMEM((_CH,), jnp.int32),
        pltpu.VMEM((_CH,), jnp.int32),
        pltpu.VMEM((_CH,), jnp.int32),
        pltpu.VMEM((_CH,), jnp.int32),
        pltpu.VMEM((_CH, _W), jnp.float32),
        pltpu.VMEM((_CH, _W), jnp.float32),
        pltpu.VMEM((_CH, _D), jnp.float32),
        pltpu.VMEM((_CH, _D), jnp.float32),
        pltpu.VMEM((_CH, 16), jnp.float32),
        pltpu.VMEM((16,), jnp.float32),
    ] + [pltpu.SemaphoreType.DMA] * 10,
)


def _table(h, xe_ref, xn_ref):
    n = jnp.sqrt(jnp.sum(h * h, axis=1, keepdims=True))
    hn = h / jnp.maximum(n, 1e-12)
    xn_ref[...] = hn
    xe_ref[...] = jnp.concatenate(
        [hn, n, jnp.zeros((h.shape[0], _W - _D - 1), jnp.float32)], axis=1)


def _pre_body(x_ref, w_ref, b_ref, xe_ref, xn_ref):
    h = jnp.dot(x_ref[...], w_ref[...], preferred_element_type=jnp.float32)
    _table(jnp.maximum(h + b_ref[...], 0.0), xe_ref, xn_ref)


_pre_call = pl.pallas_call(
    _pre_body,
    out_shape=(jax.ShapeDtypeStruct((_NP, _W), jnp.float32),
               jax.ShapeDtypeStruct((_NP, _D), jnp.float32)),
)


def _combine(acc_ref, xe_ref, beta):
    n = xe_ref[:, _D:_D + 1]
    sdot = (n / jnp.maximum(n, 1e-12)) ** 2
    es = jnp.exp(beta * sdot)
    num = (acc_ref[0, :, :_D] + acc_ref[1, :, :_D]
           + es * (n * xe_ref[:, :_D]))
    den = (acc_ref[0, :, _D:_D + 1] + acc_ref[1, :, _D:_D + 1]
           + es + 1e-16)
    return num / den


def _mid_body(acc_ref, xe_ref, xe2_ref, xn2_ref):
    _table(_combine(acc_ref, xe_ref, 1.0), xe2_ref, xn2_ref)


_MBLK = 2048

_mid_call = pl.pallas_call(
    _mid_body,
    grid=(_NP // _MBLK,),
    in_specs=[
        pl.BlockSpec((_NC, _MBLK, _W), lambda i: (0, i, 0)),
        pl.BlockSpec((_MBLK, _W), lambda i: (i, 0)),
    ],
    out_specs=(pl.BlockSpec((_MBLK, _W), lambda i: (i, 0)),
               pl.BlockSpec((_MBLK, _D), lambda i: (i, 0))),
    out_shape=(jax.ShapeDtypeStruct((_NP, _W), jnp.float32),
               jax.ShapeDtypeStruct((_NP, _D), jnp.float32)),
)


def _post_body(acc_ref, xe_ref, beta_ref, w_ref, b_ref, out_ref):
    h1 = _combine(acc_ref, xe_ref, beta_ref[0, 0])
    o = jnp.dot(h1, w_ref[...], preferred_element_type=jnp.float32) + b_ref[...]
    m = jnp.max(o, axis=1, keepdims=True)
    lse = jnp.log(jnp.sum(jnp.exp(o - m), axis=1, keepdims=True)) + m
    out_ref[...] = o - lse


_post_call = pl.pallas_call(
    _post_body,
    out_shape=jax.ShapeDtypeStruct((_NP, _DO), jnp.float32),
)


def kernel(x, edge_index, W1, b1, beta2, W2, b2):
    src = edge_index[0].astype(jnp.int32)
    dst = edge_index[1].astype(jnp.int32)
    dummy = jnp.full((_EPA - _E,), _N, jnp.int32)
    src = jnp.concatenate([src, dummy])
    dst = jnp.concatenate([dst, dummy])
    xp = jnp.zeros((_NP, _D), jnp.float32).at[:_N].set(x.astype(jnp.float32))

    z144 = jnp.zeros((_NP, _W), jnp.float32)
    beta2f = beta2.astype(jnp.float32)

    xe1, xn1 = _pre_call(xp, W1.T, b1.reshape(1, _D))
    acc1 = _sc_conv(xe1, xn1, src, dst, jnp.ones((16,), jnp.float32), z144)
    xe2, xn2 = _mid_call(acc1, xe1)
    acc2 = _sc_conv(xe2, xn2, src, dst,
                    jnp.full((16,), beta2f, jnp.float32), z144)
    out = _post_call(acc2, xe2, beta2f.reshape(1, 1), W2.T, b2.reshape(1, _DO))
    return out[:_N]
